# Initial kernel scaffold; baseline (speedup 1.0000x reference)
import jax
import jax.numpy as jnp
from jax.experimental import pallas as pl


def kernel(edge_index, fund_emb, stock_emb, W1l, b1l, W1r, W2l, b2l, W2r):
    raise NotImplementedError("write your pallas kernel here")


# speedup vs baseline: 4.2846x; 4.2846x over previous
import functools

import jax
import jax.numpy as jnp
from jax import lax
from jax.experimental import pallas as pl
from jax.experimental.pallas import tpu as pltpu
from jax.experimental.pallas import tpu_sc as plsc

N_TOTAL = 50000
N_EDGES = 800000
EMB = 64
HID = 128
OUT = 64

NC = 2
NS = 16
HALF = N_TOTAL // NC
STRIPE = 1568
HP = NS * STRIPE
N_TRASH = 64
CH = 128
EPT = N_EDGES // NS
NWIN = EPT // CH
TAIL_BASE = NS * (NWIN * CH)
TAIL = (N_EDGES - TAIL_BASE) // NS
EXP_CH = 112
N_EXP = STRIPE // EXP_CH

R_BLK = 1000
N_RB = HALF // R_BLK


def _seg_sum_body(with_count, *refs):
    if with_count:
        (src_hbm, dst_hbm, x_hbm, zrows_hbm, zcnt_hbm,
         out_acc, out_cnt,
         src_v, dst_v, rows_v, ones_v, src_t, dst_t, ebuf, cbuf,
         acc_sh, cnt_sh, sem) = refs
    else:
        (src_hbm, dst_hbm, x_hbm, zrows_hbm,
         out_acc,
         src_v, dst_v, rows_v, src_t, dst_t, ebuf,
         acc_sh, sem) = refs

    c = lax.axis_index("c")
    s = lax.axis_index("s")
    base_node = c * HALF

    pltpu.sync_copy(zrows_hbm, ebuf)

    @pl.loop(0, N_EXP)
    def _zero(k):
        pltpu.sync_copy(ebuf, acc_sh.at[pl.ds(s * STRIPE + k * EXP_CH, EXP_CH)])

    if with_count:
        pltpu.sync_copy(zcnt_hbm, cbuf)
        pltpu.sync_copy(cbuf, cnt_sh.at[pl.ds(s * STRIPE, STRIPE)])

        @pl.loop(0, CH // 16)
        def _fill_ones(i):
            ones_v[pl.ds(i * 16, 16)] = jnp.full((16,), 1.0, jnp.float32)

    plsc.subcore_barrier()

    def localize(dref, n):
        @pl.loop(0, n // 16)
        def _fix(i):
            d = dref[pl.ds(i * 16, 16)]
            local = d - base_node
            inb = (local >= 0) & (local < HALF)
            trash = HP + (d & (N_TRASH - 1))
            dref[pl.ds(i * 16, 16)] = jnp.where(inb, local, trash)

    ebase = s * (NWIN * CH)

    @pl.loop(0, NWIN)
    def _window(w):
        off = pl.multiple_of(ebase + w * CH, CH)
        pltpu.sync_copy(src_hbm.at[pl.ds(off, CH)], src_v)
        pltpu.sync_copy(dst_hbm.at[pl.ds(off, CH)], dst_v)
        pltpu.async_copy(x_hbm.at[src_v], rows_v, sem).wait()
        localize(dst_v, CH)
        pltpu.sync_copy(rows_v, acc_sh.at[dst_v], add=True)
        if with_count:
            pltpu.sync_copy(ones_v, cnt_sh.at[dst_v], add=True)

    toff = TAIL_BASE + s * TAIL
    pltpu.sync_copy(src_hbm.at[pl.ds(toff, TAIL)], src_t)
    pltpu.sync_copy(dst_hbm.at[pl.ds(toff, TAIL)], dst_t)
    pltpu.async_copy(x_hbm.at[src_t], rows_v.at[pl.ds(0, TAIL)], sem).wait()
    localize(dst_t, TAIL)
    pltpu.sync_copy(rows_v.at[pl.ds(0, TAIL)], acc_sh.at[dst_t], add=True)
    if with_count:
        pltpu.sync_copy(ones_v.at[pl.ds(0, TAIL)], cnt_sh.at[dst_t], add=True)

    plsc.subcore_barrier()

    @pl.loop(0, N_EXP)
    def _export(k):
        off = s * STRIPE + k * EXP_CH
        pltpu.sync_copy(acc_sh.at[pl.ds(off, EXP_CH)], ebuf)
        pltpu.sync_copy(ebuf, out_acc.at[c, pl.ds(off, EXP_CH)])

    if with_count:
        pltpu.sync_copy(cnt_sh.at[pl.ds(s * STRIPE, STRIPE)], cbuf)
        pltpu.sync_copy(cbuf, out_cnt.at[pl.ds(c * HP + s * STRIPE, STRIPE)])


def _make_seg_sum(with_count):
    mesh = plsc.VectorSubcoreMesh(core_axis_name="c", subcore_axis_name="s",
                                  num_cores=NC, num_subcores=NS)
    out_type = [jax.ShapeDtypeStruct((NC, HP, EMB), jnp.float32)]
    scratch = [
        pltpu.VMEM((CH,), jnp.int32),
        pltpu.VMEM((CH,), jnp.int32),
        pltpu.VMEM((CH, EMB), jnp.float32),
    ]
    if with_count:
        out_type.append(jax.ShapeDtypeStruct((NC * HP,), jnp.float32))
        scratch.append(pltpu.VMEM((CH,), jnp.float32))
    scratch += [
        pltpu.VMEM((TAIL,), jnp.int32),
        pltpu.VMEM((TAIL,), jnp.int32),
        pltpu.VMEM((EXP_CH, EMB), jnp.float32),
    ]
    if with_count:
        scratch.append(pltpu.VMEM((STRIPE,), jnp.float32))
    scratch.append(pltpu.VMEM_SHARED((HP + N_TRASH, EMB), jnp.float32))
    if with_count:
        scratch.append(pltpu.VMEM_SHARED((HP + N_TRASH,), jnp.float32))
    scratch.append(pltpu.SemaphoreType.DMA)

    return pl.kernel(
        functools.partial(_seg_sum_body, with_count),
        out_type=out_type,
        mesh=mesh,
        scratch_types=scratch,
        compiler_params=pltpu.CompilerParams(use_tc_tiling_on_sc=False),
        name="sc_segsum_cnt" if with_count else "sc_segsum",
    )


def _layer1_tc_body(acc_ref, cnt_ref, x_ref, w1l_ref, b1l_ref, w1r_ref,
                    w2l_ref, w2r_ref, b2l_ref, g_ref, hr_ref, invc_ref):
    invc = 1.0 / jnp.maximum(cnt_ref[0], 1.0)
    mean = acc_ref[0] * invc
    h = jnp.dot(mean, w1l_ref[...], preferred_element_type=jnp.float32)
    h = h + b1l_ref[...] + jnp.dot(x_ref[...], w1r_ref[...],
                                   preferred_element_type=jnp.float32)
    h = jnp.maximum(h, 0.0)
    g_ref[...] = jnp.dot(h, w2l_ref[...], preferred_element_type=jnp.float32)
    hr_ref[...] = jnp.dot(h, w2r_ref[...],
                          preferred_element_type=jnp.float32) + b2l_ref[...]
    invc_ref[0] = invc


def _layer2_tc_body(acc_ref, invc_ref, hr_ref, z_ref):
    z_ref[...] = acc_ref[0] * invc_ref[0] + hr_ref[...]


def _layer1_tc(acc1, cnt, x, W1l, b1l, W1r, W2l, W2r, b2l):
    grid = (NC, N_RB)
    return pl.pallas_call(
        _layer1_tc_body,
        grid=grid,
        in_specs=[
            pl.BlockSpec((1, R_BLK, EMB), lambda c, i: (c, i, 0)),
            pl.BlockSpec((1, R_BLK, 1), lambda c, i: (c, i, 0)),
            pl.BlockSpec((R_BLK, EMB), lambda c, i: (c * N_RB + i, 0)),
            pl.BlockSpec((EMB, HID), lambda c, i: (0, 0)),
            pl.BlockSpec((1, HID), lambda c, i: (0, 0)),
            pl.BlockSpec((EMB, HID), lambda c, i: (0, 0)),
            pl.BlockSpec((HID, OUT), lambda c, i: (0, 0)),
            pl.BlockSpec((HID, OUT), lambda c, i: (0, 0)),
            pl.BlockSpec((1, OUT), lambda c, i: (0, 0)),
        ],
        out_specs=[
            pl.BlockSpec((R_BLK, OUT), lambda c, i: (c * N_RB + i, 0)),
            pl.BlockSpec((R_BLK, OUT), lambda c, i: (c * N_RB + i, 0)),
            pl.BlockSpec((1, R_BLK, 1), lambda c, i: (c, i, 0)),
        ],
        out_shape=[
            jax.ShapeDtypeStruct((N_TOTAL, OUT), jnp.float32),
            jax.ShapeDtypeStruct((N_TOTAL, OUT), jnp.float32),
            jax.ShapeDtypeStruct((NC, HP, 1), jnp.float32),
        ],
    )(acc1, cnt, x, W1l, b1l, W1r, W2l, W2r, b2l)


def _layer2_tc(acc2, invc, hr):
    grid = (NC, N_RB)
    return pl.pallas_call(
        _layer2_tc_body,
        grid=grid,
        in_specs=[
            pl.BlockSpec((1, R_BLK, OUT), lambda c, i: (c, i, 0)),
            pl.BlockSpec((1, R_BLK, 1), lambda c, i: (c, i, 0)),
            pl.BlockSpec((R_BLK, OUT), lambda c, i: (c * N_RB + i, 0)),
        ],
        out_specs=pl.BlockSpec((R_BLK, OUT), lambda c, i: (c * N_RB + i, 0)),
        out_shape=jax.ShapeDtypeStruct((N_TOTAL, OUT), jnp.float32),
    )(acc2, invc, hr)


_seg_sum_cnt = _make_seg_sum(True)
_seg_sum = _make_seg_sum(False)


def kernel(edge_index, fund_emb, stock_emb, W1l, b1l, W1r, W2l, b2l, W2r):
    x = jnp.concatenate([fund_emb, stock_emb], axis=0)
    src = edge_index[0]
    dst = edge_index[1]
    zrows = jnp.zeros((EXP_CH, EMB), jnp.float32)
    zcnt = jnp.zeros((STRIPE,), jnp.float32)

    acc1, cnt = _seg_sum_cnt(src, dst, x, zrows, zcnt)
    g, hr, invc = _layer1_tc(acc1, cnt.reshape(NC, HP, 1), x,
                             W1l, b1l.reshape(1, HID), W1r,
                             W2l, W2r, b2l.reshape(1, OUT))
    (acc2,) = _seg_sum(src, dst, g, zrows)
    z = _layer2_tc(acc2, invc, hr)
    return z


# speedup vs baseline: 9.2954x; 2.1695x over previous
import functools

import jax
import jax.numpy as jnp
from jax import lax
from jax.experimental import pallas as pl
from jax.experimental.pallas import tpu as pltpu
from jax.experimental.pallas import tpu_sc as plsc

N_TOTAL = 50000
N_EDGES = 800000
EMB = 64
HID = 128
OUT = 64

NC = 2
NS = 16
HALF = N_TOTAL // NC
STRIPE = 1568
HP = NS * STRIPE
N_TRASH = 64
CH = 128
EPT = N_EDGES // NS
NWIN = EPT // CH
TAIL_BASE = NS * (NWIN * CH)
TAIL = (N_EDGES - TAIL_BASE) // NS
EXP_CH = 112
N_EXP = STRIPE // EXP_CH

R_BLK = 1000
N_RB = HALF // R_BLK


NSLOT = 3


def _seg_sum_body(with_count, *refs):
    if with_count:
        (src_hbm, dst_hbm, x_hbm, zrows_hbm, zcnt_hbm,
         out_acc, out_cnt,
         s0, s1, s2, d0, d1, d2, r0, r1, r2, ones_v, src_t, dst_t,
         acc_sh, cnt_sh,
         i0, i1, i2, g0, g1, g2, a0, a1, a2, c0, c1, c2, sem) = refs
        csem = (c0, c1, c2)
    else:
        (src_hbm, dst_hbm, x_hbm, zrows_hbm,
         out_acc,
         s0, s1, s2, d0, d1, d2, r0, r1, r2, src_t, dst_t,
         acc_sh,
         i0, i1, i2, g0, g1, g2, a0, a1, a2, sem) = refs
    srcs = (s0, s1, s2)
    dsts = (d0, d1, d2)
    rows = (r0, r1, r2)
    isem = (i0, i1, i2)
    gsem = (g0, g1, g2)
    ssem = (a0, a1, a2)

    c = lax.axis_index("c")
    s = lax.axis_index("s")
    base_node = c * HALF

    pltpu.sync_copy(zrows_hbm, rows[0].at[pl.ds(0, EXP_CH)])

    @pl.loop(0, N_EXP)
    def _zero(k):
        pltpu.sync_copy(rows[0].at[pl.ds(0, EXP_CH)],
                        acc_sh.at[pl.ds(s * STRIPE + k * EXP_CH, EXP_CH)])

    if with_count:
        pltpu.sync_copy(zcnt_hbm, ones_v.at[pl.ds(0, EXP_CH)])

        @pl.loop(0, N_EXP)
        def _zcnt(k):
            pltpu.sync_copy(ones_v.at[pl.ds(0, EXP_CH)],
                            cnt_sh.at[pl.ds(s * STRIPE + k * EXP_CH, EXP_CH)])

        @pl.loop(0, CH // 16)
        def _fill_ones(i):
            ones_v[pl.ds(i * 16, 16)] = jnp.full((16,), 1.0, jnp.float32)

    plsc.subcore_barrier()

    def localize(dref, n):
        @pl.loop(0, n // 16)
        def _fix(i):
            d = dref[pl.ds(i * 16, 16)]
            local = d - base_node
            inb = (local >= 0) & (local < HALF)
            trash = HP + (d & (N_TRASH - 1))
            dref[pl.ds(i * 16, 16)] = jnp.where(inb, local, trash)

    ebase = s * (NWIN * CH)

    def idx_start(w, b):
        off = pl.multiple_of(ebase + w * CH, 8)
        pltpu.async_copy(src_hbm.at[pl.ds(off, CH)], srcs[b], isem[b])
        pltpu.async_copy(dst_hbm.at[pl.ds(off, CH)], dsts[b], isem[b])

    def idx_wait(w, b):
        off = pl.multiple_of(ebase + w * CH, 8)
        pltpu.make_async_copy(src_hbm.at[pl.ds(off, CH)], srcs[b], isem[b]).wait()
        pltpu.make_async_copy(dst_hbm.at[pl.ds(off, CH)], dsts[b], isem[b]).wait()

    def gather_start(b):
        pltpu.async_copy(x_hbm.at[srcs[b]], rows[b], gsem[b])

    def gather_wait(b):
        pltpu.make_async_copy(x_hbm.at[srcs[b]], rows[b], gsem[b]).wait()

    def scatter_start(b):
        pltpu.async_copy(rows[b], acc_sh.at[dsts[b]], ssem[b], add=True)
        if with_count:
            pltpu.async_copy(ones_v, cnt_sh.at[dsts[b]], csem[b], add=True)

    def scatter_wait(b):
        pltpu.make_async_copy(rows[b], acc_sh.at[dsts[b]], ssem[b]).wait()
        if with_count:
            pltpu.make_async_copy(ones_v, cnt_sh.at[dsts[b]], csem[b]).wait()

    idx_start(0, 0)

    @pl.loop(0, NWIN, step=NSLOT)
    def _window(w0):
        for d in range(NSLOT):
            w = w0 + d
            b = d
            p = (d + NSLOT - 1) % NSLOT
            q = (d + 1) % NSLOT
            idx_wait(w, b)
            localize(dsts[b], CH)
            gather_start(b)

            @pl.when(w >= 1)
            def _prev():
                gather_wait(p)
                scatter_start(p)

            @pl.when(w >= 2)
            def _drain():
                scatter_wait(q)

            idx_start(w + 1, q)

    b_last = (NWIN - 1) % NSLOT
    gather_wait(b_last)
    scatter_start(b_last)
    scatter_wait((NWIN - 2) % NSLOT)
    scatter_wait(b_last)
    idx_wait(NWIN, NWIN % NSLOT)

    toff = TAIL_BASE + s * TAIL
    pltpu.sync_copy(src_hbm.at[pl.ds(toff, TAIL)], src_t)
    pltpu.sync_copy(dst_hbm.at[pl.ds(toff, TAIL)], dst_t)
    pltpu.async_copy(x_hbm.at[src_t], rows[0].at[pl.ds(0, TAIL)], sem).wait()
    localize(dst_t, TAIL)
    pltpu.sync_copy(rows[0].at[pl.ds(0, TAIL)], acc_sh.at[dst_t], add=True)
    if with_count:
        pltpu.sync_copy(ones_v.at[pl.ds(0, TAIL)], cnt_sh.at[dst_t], add=True)

    plsc.subcore_barrier()

    @pl.loop(0, N_EXP)
    def _export(k):
        off = s * STRIPE + k * EXP_CH
        pltpu.sync_copy(acc_sh.at[pl.ds(off, EXP_CH)], rows[0].at[pl.ds(0, EXP_CH)])
        pltpu.sync_copy(rows[0].at[pl.ds(0, EXP_CH)], out_acc.at[c, pl.ds(off, EXP_CH)])

    if with_count:
        @pl.loop(0, N_EXP)
        def _ecnt(k):
            off = s * STRIPE + k * EXP_CH
            pltpu.sync_copy(cnt_sh.at[pl.ds(off, EXP_CH)], ones_v.at[pl.ds(0, EXP_CH)])
            pltpu.sync_copy(ones_v.at[pl.ds(0, EXP_CH)],
                            out_cnt.at[pl.ds(c * HP + off, EXP_CH)])


def _make_seg_sum(with_count):
    mesh = plsc.VectorSubcoreMesh(core_axis_name="c", subcore_axis_name="s",
                                  num_cores=NC, num_subcores=NS)
    out_type = [jax.ShapeDtypeStruct((NC, HP, EMB), jnp.float32)]
    scratch = (
        [pltpu.VMEM((CH,), jnp.int32) for _ in range(NSLOT)] +
        [pltpu.VMEM((CH,), jnp.int32) for _ in range(NSLOT)] +
        [pltpu.VMEM((CH, EMB), jnp.float32) for _ in range(NSLOT)]
    )
    if with_count:
        out_type.append(jax.ShapeDtypeStruct((NC * HP,), jnp.float32))
        scratch.append(pltpu.VMEM((CH,), jnp.float32))
    scratch += [
        pltpu.VMEM((TAIL,), jnp.int32),
        pltpu.VMEM((TAIL,), jnp.int32),
    ]
    scratch.append(pltpu.VMEM_SHARED((HP + N_TRASH, EMB), jnp.float32))
    if with_count:
        scratch.append(pltpu.VMEM_SHARED((HP + N_TRASH,), jnp.float32))
    n_sem = 4 * NSLOT if with_count else 3 * NSLOT
    scratch += [pltpu.SemaphoreType.DMA for _ in range(n_sem)]
    scratch.append(pltpu.SemaphoreType.DMA)

    return pl.kernel(
        functools.partial(_seg_sum_body, with_count),
        out_type=out_type,
        mesh=mesh,
        scratch_types=scratch,
        compiler_params=pltpu.CompilerParams(use_tc_tiling_on_sc=False),
        name="sc_segsum_cnt" if with_count else "sc_segsum",
    )


def _layer1_tc_body(acc_ref, cnt_ref, x_ref, w1l_ref, b1l_ref, w1r_ref,
                    w2l_ref, w2r_ref, b2l_ref, g_ref, hr_ref, invc_ref):
    invc = 1.0 / jnp.maximum(cnt_ref[0], 1.0)
    mean = acc_ref[0] * invc
    h = jnp.dot(mean, w1l_ref[...], preferred_element_type=jnp.float32)
    h = h + b1l_ref[...] + jnp.dot(x_ref[...], w1r_ref[...],
                                   preferred_element_type=jnp.float32)
    h = jnp.maximum(h, 0.0)
    g_ref[...] = jnp.dot(h, w2l_ref[...], preferred_element_type=jnp.float32)
    hr_ref[...] = jnp.dot(h, w2r_ref[...],
                          preferred_element_type=jnp.float32) + b2l_ref[...]
    invc_ref[0] = invc


def _layer2_tc_body(acc_ref, invc_ref, hr_ref, z_ref):
    z_ref[...] = acc_ref[0] * invc_ref[0] + hr_ref[...]


def _layer1_tc(acc1, cnt, x, W1l, b1l, W1r, W2l, W2r, b2l):
    grid = (NC, N_RB)
    return pl.pallas_call(
        _layer1_tc_body,
        grid=grid,
        in_specs=[
            pl.BlockSpec((1, R_BLK, EMB), lambda c, i: (c, i, 0)),
            pl.BlockSpec((1, R_BLK, 1), lambda c, i: (c, i, 0)),
            pl.BlockSpec((R_BLK, EMB), lambda c, i: (c * N_RB + i, 0)),
            pl.BlockSpec((EMB, HID), lambda c, i: (0, 0)),
            pl.BlockSpec((1, HID), lambda c, i: (0, 0)),
            pl.BlockSpec((EMB, HID), lambda c, i: (0, 0)),
            pl.BlockSpec((HID, OUT), lambda c, i: (0, 0)),
            pl.BlockSpec((HID, OUT), lambda c, i: (0, 0)),
            pl.BlockSpec((1, OUT), lambda c, i: (0, 0)),
        ],
        out_specs=[
            pl.BlockSpec((R_BLK, OUT), lambda c, i: (c * N_RB + i, 0)),
            pl.BlockSpec((R_BLK, OUT), lambda c, i: (c * N_RB + i, 0)),
            pl.BlockSpec((1, R_BLK, 1), lambda c, i: (c, i, 0)),
        ],
        out_shape=[
            jax.ShapeDtypeStruct((N_TOTAL, OUT), jnp.float32),
            jax.ShapeDtypeStruct((N_TOTAL, OUT), jnp.float32),
            jax.ShapeDtypeStruct((NC, HP, 1), jnp.float32),
        ],
    )(acc1, cnt, x, W1l, b1l, W1r, W2l, W2r, b2l)


def _layer2_tc(acc2, invc, hr):
    grid = (NC, N_RB)
    return pl.pallas_call(
        _layer2_tc_body,
        grid=grid,
        in_specs=[
            pl.BlockSpec((1, R_BLK, OUT), lambda c, i: (c, i, 0)),
            pl.BlockSpec((1, R_BLK, 1), lambda c, i: (c, i, 0)),
            pl.BlockSpec((R_BLK, OUT), lambda c, i: (c * N_RB + i, 0)),
        ],
        out_specs=pl.BlockSpec((R_BLK, OUT), lambda c, i: (c * N_RB + i, 0)),
        out_shape=jax.ShapeDtypeStruct((N_TOTAL, OUT), jnp.float32),
    )(acc2, invc, hr)


_seg_sum_cnt = _make_seg_sum(True)
_seg_sum = _make_seg_sum(False)


def kernel(edge_index, fund_emb, stock_emb, W1l, b1l, W1r, W2l, b2l, W2r):
    x = jnp.concatenate([fund_emb, stock_emb], axis=0)
    src = edge_index[0]
    dst = edge_index[1]
    zrows = jnp.zeros((EXP_CH, EMB), jnp.float32)
    zcnt = jnp.zeros((EXP_CH,), jnp.float32)

    acc1, cnt = _seg_sum_cnt(src, dst, x, zrows, zcnt)
    g, hr, invc = _layer1_tc(acc1, cnt.reshape(NC, HP, 1), x,
                             W1l, b1l.reshape(1, HID), W1r,
                             W2l, W2r, b2l.reshape(1, OUT))
    (acc2,) = _seg_sum(src, dst, g, zrows)
    z = _layer2_tc(acc2, invc, hr)
    return z
